# baseline (speedup 1.0000x reference)
import jax
import jax.numpy as jnp
from jax.experimental import pallas as pl
from jax.experimental.pallas import tpu as pltpu

THRESHOLD = 0.05
WEIGHT = 1.0

B = 8
N = 16384
M = 2048
BN = 2048
NBLK = N // BN


def _loss_kernel(a_ref, pt_ref, out_ref, acc_ref):
    b = pl.program_id(0)
    i = pl.program_id(1)
    step = b * NBLK + i

    @pl.when(step == 0)
    def _init():
        acc_ref[0] = 0.0
        acc_ref[1] = 0.0

    a = a_ref[0]
    pt = pt_ref[0]

    CH = 256
    acc = None
    for k in range(M // CH):
        ptc = pt[:, k * CH:(k + 1) * CH]
        ec = jnp.dot(a, ptc, preferred_element_type=jnp.float32)
        ec = ec + ptc[4:5, :]
        e3 = ec.reshape(BN, CH // 128, 128)
        cmin = jnp.minimum(e3[:, 0], e3[:, 1])
        acc = cmin if acc is None else jnp.minimum(acc, cmin)
    m = jnp.min(acc, axis=1)

    dmin = jnp.maximum(m + a[:, 3], 0.0)
    mask = dmin < THRESHOLD
    acc_ref[0] += jnp.sum(jnp.where(mask, dmin, 0.0))
    acc_ref[1] += jnp.sum(mask.astype(jnp.float32))

    @pl.when(step == B * NBLK - 1)
    def _finish():
        s = acc_ref[0]
        mm = acc_ref[1]
        out_ref[0, 0] = jnp.where(mm > 0.0, s / (mm + 1e-6), 0.0)


@jax.jit
def kernel(completed, partial):
    c2 = jnp.sum(completed * completed, axis=-1, keepdims=True)
    a = jnp.concatenate([-2.0 * completed, c2], axis=-1)
    a = jnp.pad(a, ((0, 0), (0, 0), (0, 4)))

    p2 = jnp.sum(partial * partial, axis=-1, keepdims=True)
    zero_p = jnp.zeros_like(p2)
    paug = jnp.concatenate([partial, zero_p, p2], axis=-1)
    paug = jnp.pad(paug, ((0, 0), (0, 0), (0, 3)))
    pt = jnp.transpose(paug, (0, 2, 1))

    out = pl.pallas_call(
        _loss_kernel,
        grid=(B, NBLK),
        in_specs=[
            pl.BlockSpec((1, BN, 8), lambda b, i: (b, i, 0)),
            pl.BlockSpec((1, 8, M), lambda b, i: (b, 0, 0)),
        ],
        out_specs=pl.BlockSpec(memory_space=pltpu.SMEM),
        out_shape=jax.ShapeDtypeStruct((1, 1), jnp.float32),
        scratch_shapes=[pltpu.SMEM((2,), jnp.float32)],
    )(a, pt)
    return WEIGHT * out[0, 0]


# speedup vs baseline: 5.5417x; 5.5417x over previous
import jax
import jax.numpy as jnp
from jax.experimental import pallas as pl
from jax.experimental.pallas import tpu as pltpu

THRESHOLD = 0.05
WEIGHT = 1.0

B = 8
N = 16384
M = 2048
BN = 2048
NBLK = N // BN
NCH = BN // 128


def _loss_kernel(pg_ref, at_ref, p2b_ref, c2l_ref, out_ref, s_ref, n_ref):
    b = pl.program_id(0)
    i = pl.program_id(1)
    step = b * NBLK + i

    @pl.when(step == 0)
    def _init():
        s_ref[...] = jnp.zeros_like(s_ref)
        n_ref[...] = jnp.zeros_like(n_ref)

    pg = pg_ref[0]
    at = at_ref[0]
    p2b = p2b_ref[0]
    c2l = c2l_ref[0]

    svec = jnp.zeros((1, 128), jnp.float32)
    nvec = jnp.zeros((1, 128), jnp.float32)
    for q in range(NCH):
        atc = at[:, q * 128:(q + 1) * 128]
        e = jnp.dot(pg, atc, preferred_element_type=jnp.float32)
        e = e + p2b
        rows = M
        while rows > 8:
            half = rows // 2
            e = jnp.minimum(e[:half], e[half:rows])
            rows = half
        dmin8 = e
        dminc = jnp.min(dmin8, axis=0, keepdims=True)
        dminc = jnp.maximum(dminc + c2l[q:q + 1, :], 0.0)
        mask = dminc < THRESHOLD
        svec = svec + jnp.where(mask, dminc, 0.0)
        nvec = nvec + mask.astype(jnp.float32)

    s_ref[...] += svec
    n_ref[...] += nvec

    @pl.when(step == B * NBLK - 1)
    def _finish():
        s = jnp.sum(s_ref[...])
        mm = jnp.sum(n_ref[...])
        out_ref[0, 0] = jnp.where(mm > 0.0, s / (mm + 1e-6), 0.0)


@jax.jit
def kernel(completed, partial):
    pg = jnp.pad(partial, ((0, 0), (0, 0), (0, 5)))
    at = jnp.transpose(-2.0 * completed, (0, 2, 1))
    at = jnp.pad(at, ((0, 0), (0, 5), (0, 0)))

    p2 = jnp.sum(partial * partial, axis=-1, keepdims=True)
    p2b = jnp.broadcast_to(p2, (B, M, 128))
    c2 = jnp.sum(completed * completed, axis=-1)
    c2l = c2.reshape(B, N // 128, 128)

    out = pl.pallas_call(
        _loss_kernel,
        grid=(B, NBLK),
        in_specs=[
            pl.BlockSpec((1, M, 8), lambda b, i: (b, 0, 0)),
            pl.BlockSpec((1, 8, BN), lambda b, i: (b, 0, i)),
            pl.BlockSpec((1, M, 128), lambda b, i: (b, 0, 0)),
            pl.BlockSpec((1, NCH, 128), lambda b, i: (b, i, 0)),
        ],
        out_specs=pl.BlockSpec(memory_space=pltpu.SMEM),
        out_shape=jax.ShapeDtypeStruct((1, 1), jnp.float32),
        scratch_shapes=[
            pltpu.VMEM((1, 128), jnp.float32),
            pltpu.VMEM((1, 128), jnp.float32),
        ],
    )(pg, at, p2b, c2l)
    return WEIGHT * out[0, 0]


# speedup vs baseline: 5.6802x; 1.0250x over previous
import jax
import jax.numpy as jnp
from jax.experimental import pallas as pl
from jax.experimental.pallas import tpu as pltpu

THRESHOLD = 0.05
WEIGHT = 1.0

B = 8
N = 16384
M = 2048
BN = 4096
NBLK = N // BN
CH = 256
NCH = BN // CH


def _loss_kernel(pg_ref, at_ref, p2b_ref, c2l_ref, out_ref, s_ref, n_ref):
    b = pl.program_id(0)
    i = pl.program_id(1)
    step = b * NBLK + i

    @pl.when(step == 0)
    def _init():
        s_ref[...] = jnp.zeros_like(s_ref)
        n_ref[...] = jnp.zeros_like(n_ref)

    pg = pg_ref[0]
    at = at_ref[0]
    p2b = p2b_ref[0]
    c2l = c2l_ref[0]

    svec = jnp.zeros((1, CH), jnp.float32)
    nvec = jnp.zeros((1, CH), jnp.float32)
    for q in range(NCH):
        atc = at[:, q * CH:(q + 1) * CH]
        e = jnp.dot(pg, atc, preferred_element_type=jnp.float32)
        e = e + p2b
        rows = M
        while rows > 8:
            half = rows // 2
            e = jnp.minimum(e[:half], e[half:rows])
            rows = half
        dmin8 = e
        dminc = jnp.min(dmin8, axis=0, keepdims=True)
        dminc = jnp.maximum(dminc + c2l[q:q + 1, :], 0.0)
        mask = dminc < THRESHOLD
        svec = svec + jnp.where(mask, dminc, 0.0)
        nvec = nvec + mask.astype(jnp.float32)

    s_ref[...] += svec
    n_ref[...] += nvec

    @pl.when(step == B * NBLK - 1)
    def _finish():
        s = jnp.sum(s_ref[...])
        mm = jnp.sum(n_ref[...])
        out_ref[0, 0] = jnp.where(mm > 0.0, s / (mm + 1e-6), 0.0)


@jax.jit
def kernel(completed, partial):
    pg = jnp.pad(partial, ((0, 0), (0, 0), (0, 5)))
    at = jnp.transpose(-2.0 * completed, (0, 2, 1))
    at = jnp.pad(at, ((0, 0), (0, 5), (0, 0)))

    p2 = jnp.sum(partial * partial, axis=-1, keepdims=True)
    p2b = jnp.broadcast_to(p2, (B, M, CH))
    c2 = jnp.sum(completed * completed, axis=-1)
    c2l = c2.reshape(B, N // CH, CH)

    out = pl.pallas_call(
        _loss_kernel,
        grid=(B, NBLK),
        in_specs=[
            pl.BlockSpec((1, M, 8), lambda b, i: (b, 0, 0)),
            pl.BlockSpec((1, 8, BN), lambda b, i: (b, 0, i)),
            pl.BlockSpec((1, M, CH), lambda b, i: (b, 0, 0)),
            pl.BlockSpec((1, NCH, CH), lambda b, i: (b, i, 0)),
        ],
        out_specs=pl.BlockSpec(memory_space=pltpu.SMEM),
        out_shape=jax.ShapeDtypeStruct((1, 1), jnp.float32),
        scratch_shapes=[
            pltpu.VMEM((1, CH), jnp.float32),
            pltpu.VMEM((1, CH), jnp.float32),
        ],
    )(pg, at, p2b, c2l)
    return WEIGHT * out[0, 0]
